# baseline (speedup 1.0000x reference)
import functools

import jax
import jax.numpy as jnp
from jax import lax
from jax.experimental import pallas as pl
from jax.experimental.pallas import tpu as pltpu
from jax.experimental.pallas import tpu_sc as plsc

B = 4096
L = 200
EMBED = 128
SUB = 128
NUM_CLASSES = 10000

NC, NS = 2, 16
NW = NC * NS
CL0 = 104
CL1 = 96
NCH = 2
VEC = 16

NBUF = 4
NSPLIT = 8
BS = B // NSPLIT


def _sc_pool(table, idxa, idxb, bs):
    rpw = bs // NW
    nchk = rpw * NCH
    ngrp = nchk // NBUF
    mesh = plsc.VectorSubcoreMesh(core_axis_name="c", subcore_axis_name="s")

    @functools.partial(
        pl.kernel,
        out_type=jax.ShapeDtypeStruct((bs, EMBED), jnp.float32),
        mesh=mesh,
        scratch_types=[
            pltpu.VMEM((rpw, CL0), jnp.int32),
            pltpu.VMEM((rpw, CL1), jnp.int32),
            pltpu.VMEM((NBUF // 2, CL0, EMBED), jnp.float32),
            pltpu.VMEM((NBUF // 2, CL1, EMBED), jnp.float32),
            pltpu.VMEM((rpw, EMBED), jnp.float32),
            [pltpu.SemaphoreType.DMA] * NBUF,
        ],
    )
    def k(table_hbm, idxa_hbm, idxb_hbm, out_hbm, idxa_v, idxb_v,
          gbufa, gbufb, obuf, sems):
        wid = lax.axis_index("s") * NC + lax.axis_index("c")
        base = wid * rpw
        pltpu.sync_copy(idxa_hbm.at[pl.ds(base, rpw)], idxa_v)
        pltpu.sync_copy(idxb_hbm.at[pl.ds(base, rpw)], idxb_v)

        def src(t, c):
            r = t // 2
            if c == 0:
                return table_hbm.at[idxa_v.at[r]]
            return table_hbm.at[idxb_v.at[r]]

        def dstbuf(b):
            c = b % 2
            return (gbufa if c == 0 else gbufb).at[b // 2]

        def add_row_a(hb, j, acc):
            return tuple(
                acc[kk] + gbufa[hb, j, pl.ds(VEC * kk, VEC)]
                for kk in range(EMBED // VEC)
            )

        def add_row_b(hb, j, acc):
            return tuple(
                acc[kk] + gbufb[hb, j, pl.ds(VEC * kk, VEC)]
                for kk in range(EMBED // VEC)
            )

        for b in range(NBUF):
            pltpu.async_copy(src(jnp.int32(b), b % 2), dstbuf(b), sems[b])

        zeros = tuple(jnp.zeros((VEC,), jnp.float32)
                      for _ in range(EMBED // VEC))
        scale = jnp.float32(1.0 / L)

        def group_body(g, acc):
            for b in range(NBUF):
                t = g * NBUF + b
                c = b % 2
                pltpu.make_async_copy(src(t, c), dstbuf(b), sems[b]).wait()
                if c == 0:
                    acc = lax.fori_loop(
                        0, CL0, functools.partial(add_row_a, b // 2), acc
                    )
                else:
                    acc = lax.fori_loop(
                        0, CL1, functools.partial(add_row_b, b // 2), acc
                    )
                    r = g * (NBUF // NCH) + b // NCH
                    for kk in range(EMBED // VEC):
                        obuf[r, pl.ds(VEC * kk, VEC)] = acc[kk] * scale
                    acc = zeros
                nxt = t + NBUF

                @pl.when(nxt < nchk)
                def _():
                    pltpu.async_copy(src(nxt, c), dstbuf(b), sems[b])
            return acc

        lax.fori_loop(0, ngrp, group_body, zeros)
        pltpu.sync_copy(obuf, out_hbm.at[pl.ds(base, rpw)])

    return k(table, idxa, idxb)


BM = 512
BN = 1024


def _mm_kernel(p_ref, s_ref, w_ref, b_ref, o_ref):
    p = p_ref[...].astype(jnp.bfloat16)
    s = s_ref[...].astype(jnp.bfloat16)
    w = w_ref[...].astype(jnp.bfloat16)
    dn = (((1,), (1,)), ((), ()))
    acc = lax.dot_general(p, w[:, :EMBED], dn,
                          preferred_element_type=jnp.float32)
    acc = acc + lax.dot_general(s, w[:, EMBED:], dn,
                                preferred_element_type=jnp.float32)
    o_ref[...] = acc + b_ref[...]


def _tc_classify(pooled, sub, W_cls, b_cls, bs):
    grid = (bs // BM, pl.cdiv(NUM_CLASSES, BN))
    return pl.pallas_call(
        _mm_kernel,
        grid=grid,
        in_specs=[
            pl.BlockSpec((BM, EMBED), lambda i, j: (i, 0)),
            pl.BlockSpec((BM, SUB), lambda i, j: (i, 0)),
            pl.BlockSpec((BN, EMBED + SUB), lambda i, j: (j, 0)),
            pl.BlockSpec((1, BN), lambda i, j: (0, j)),
        ],
        out_specs=pl.BlockSpec((BM, BN), lambda i, j: (i, j)),
        out_shape=jax.ShapeDtypeStruct((bs, NUM_CLASSES), jnp.float32),
        compiler_params=pltpu.CompilerParams(
            dimension_semantics=("parallel", "parallel"),
        ),
    )(pooled, sub, W_cls, b_cls.reshape(1, NUM_CLASSES))


def kernel(word_input, sub_category_input, table, W_cls, b_cls):
    idx = word_input.astype(jnp.int32)
    idxa = idx[:, :CL0]
    idxb = idx[:, CL0:]
    parts = []
    for k in range(NSPLIT):
        ia = lax.slice_in_dim(idxa, k * BS, (k + 1) * BS)
        ib = lax.slice_in_dim(idxb, k * BS, (k + 1) * BS)
        parts.append(_sc_pool(table, ia, ib, BS))
    pooled = jnp.concatenate(parts, axis=0)
    return _tc_classify(pooled, sub_category_input, W_cls, b_cls, B)


# speedup vs baseline: 1.0703x; 1.0703x over previous
import functools

import jax
import jax.numpy as jnp
from jax import lax
from jax.experimental import pallas as pl
from jax.experimental.pallas import tpu as pltpu
from jax.experimental.pallas import tpu_sc as plsc

B = 4096
L = 200
EMBED = 128
SUB = 128
NUM_CLASSES = 10000

NC, NS = 2, 16
NW = NC * NS
CL0 = 104
CL1 = 96
NCH = 2
VEC = 16

NBUF = 4
NSPLIT = 4
BS = B // NSPLIT


def _sc_pool(table, idxa, idxb, bs):
    rpw = bs // NW
    nchk = rpw * NCH
    ngrp = nchk // NBUF
    mesh = plsc.VectorSubcoreMesh(core_axis_name="c", subcore_axis_name="s")

    @functools.partial(
        pl.kernel,
        out_type=jax.ShapeDtypeStruct((bs, EMBED), jnp.float32),
        mesh=mesh,
        scratch_types=[
            pltpu.VMEM((rpw, CL0), jnp.int32),
            pltpu.VMEM((rpw, CL1), jnp.int32),
            pltpu.VMEM((NBUF // 2, CL0, EMBED), jnp.float32),
            pltpu.VMEM((NBUF // 2, CL1, EMBED), jnp.float32),
            pltpu.VMEM((rpw, EMBED), jnp.float32),
            [pltpu.SemaphoreType.DMA] * NBUF,
        ],
    )
    def k(table_hbm, idxa_hbm, idxb_hbm, out_hbm, idxa_v, idxb_v,
          gbufa, gbufb, obuf, sems):
        wid = lax.axis_index("s") * NC + lax.axis_index("c")
        base = wid * rpw
        pltpu.sync_copy(idxa_hbm.at[pl.ds(base, rpw)], idxa_v)
        pltpu.sync_copy(idxb_hbm.at[pl.ds(base, rpw)], idxb_v)

        def src(t, c):
            r = t // 2
            if c == 0:
                return table_hbm.at[idxa_v.at[r]]
            return table_hbm.at[idxb_v.at[r]]

        def dstbuf(b):
            c = b % 2
            return (gbufa if c == 0 else gbufb).at[b // 2]

        def add_row_a(hb, j, acc):
            return tuple(
                acc[kk] + gbufa[hb, j, pl.ds(VEC * kk, VEC)]
                for kk in range(EMBED // VEC)
            )

        def add_row_b(hb, j, acc):
            return tuple(
                acc[kk] + gbufb[hb, j, pl.ds(VEC * kk, VEC)]
                for kk in range(EMBED // VEC)
            )

        for b in range(NBUF):
            pltpu.async_copy(src(jnp.int32(b), b % 2), dstbuf(b), sems[b])

        zeros = tuple(jnp.zeros((VEC,), jnp.float32)
                      for _ in range(EMBED // VEC))
        scale = jnp.float32(1.0 / L)

        def group_body(g, acc):
            for b in range(NBUF):
                t = g * NBUF + b
                c = b % 2
                pltpu.make_async_copy(src(t, c), dstbuf(b), sems[b]).wait()
                if c == 0:
                    acc = lax.fori_loop(
                        0, CL0, functools.partial(add_row_a, b // 2), acc
                    )
                else:
                    acc = lax.fori_loop(
                        0, CL1, functools.partial(add_row_b, b // 2), acc
                    )
                    r = g * (NBUF // NCH) + b // NCH
                    for kk in range(EMBED // VEC):
                        obuf[r, pl.ds(VEC * kk, VEC)] = acc[kk] * scale
                    acc = zeros
                nxt = t + NBUF

                @pl.when(nxt < nchk)
                def _():
                    pltpu.async_copy(src(nxt, c), dstbuf(b), sems[b])
            return acc

        lax.fori_loop(0, ngrp, group_body, zeros)
        pltpu.sync_copy(obuf, out_hbm.at[pl.ds(base, rpw)])

    return k(table, idxa, idxb)


BM = 512
BN = 1024


def _mm_kernel(p_ref, s_ref, w_ref, b_ref, o_ref):
    p = p_ref[...].astype(jnp.bfloat16)
    s = s_ref[...].astype(jnp.bfloat16)
    w = w_ref[...].astype(jnp.bfloat16)
    dn = (((1,), (1,)), ((), ()))
    acc = lax.dot_general(p, w[:, :EMBED], dn,
                          preferred_element_type=jnp.float32)
    acc = acc + lax.dot_general(s, w[:, EMBED:], dn,
                                preferred_element_type=jnp.float32)
    o_ref[...] = acc + b_ref[...]


def _tc_classify(pooled, sub, W_cls, b_cls, bs):
    grid = (bs // BM, pl.cdiv(NUM_CLASSES, BN))
    return pl.pallas_call(
        _mm_kernel,
        grid=grid,
        in_specs=[
            pl.BlockSpec((BM, EMBED), lambda i, j: (i, 0)),
            pl.BlockSpec((BM, SUB), lambda i, j: (i, 0)),
            pl.BlockSpec((BN, EMBED + SUB), lambda i, j: (j, 0)),
            pl.BlockSpec((1, BN), lambda i, j: (0, j)),
        ],
        out_specs=pl.BlockSpec((BM, BN), lambda i, j: (i, j)),
        out_shape=jax.ShapeDtypeStruct((bs, NUM_CLASSES), jnp.float32),
        compiler_params=pltpu.CompilerParams(
            dimension_semantics=("parallel", "parallel"),
        ),
    )(pooled, sub, W_cls, b_cls.reshape(1, NUM_CLASSES))


def kernel(word_input, sub_category_input, table, W_cls, b_cls):
    idx = word_input.astype(jnp.int32)
    idxa = idx[:, :CL0]
    idxb = idx[:, CL0:]
    parts = []
    for k in range(NSPLIT):
        ia = lax.slice_in_dim(idxa, k * BS, (k + 1) * BS)
        ib = lax.slice_in_dim(idxb, k * BS, (k + 1) * BS)
        parts.append(_sc_pool(table, ia, ib, BS))
    pooled = jnp.concatenate(parts, axis=0)
    return _tc_classify(pooled, sub_category_input, W_cls, b_cls, B)
